# Initial kernel scaffold; baseline (speedup 1.0000x reference)
import jax
import jax.numpy as jnp
from jax.experimental import pallas as pl


def kernel(cls_out_0, cls_out_1, cls_out_2, reg_out_0, reg_out_1, reg_out_2, image_size):
    raise NotImplementedError("write your pallas kernel here")


# speedup vs baseline: 38.4337x; 38.4337x over previous
import numpy as np
import jax
import jax.numpy as jnp
from jax.experimental import pallas as pl
from jax.experimental.pallas import tpu as pltpu

_STRIDES = (8, 16, 32)
_RATIOS = np.array([0.5, 1.0, 2.0], dtype=np.float32)
_SCALE = 8.0
_NUM_PRE_NMS = 2000
_MAX_PER_IMG = 1000
_NMS_THR = 0.7
_MAX_RATIO = float(np.log(1000.0 / 16.0))
_C = 128
_INTERPRET = False


def _grid_anchors_np(H, W, stride):
    hr = np.sqrt(_RATIOS)
    wr = 1.0 / hr
    ws = stride * _SCALE * wr
    hs = stride * _SCALE * hr
    base = np.stack([-ws / 2.0, -hs / 2.0, ws / 2.0, hs / 2.0], axis=1).astype(np.float32)
    sx = np.arange(W, dtype=np.float32) * stride
    sy = np.arange(H, dtype=np.float32) * stride
    yy, xx = np.meshgrid(sy, sx, indexing='ij')
    shifts = np.stack([xx.ravel(), yy.ravel(), xx.ravel(), yy.ravel()], axis=1)
    return (shifts[:, None, :] + base[None, :, :]).reshape(-1, 4).astype(np.float32)


def _nms_kernel(ax1, ay1, ax2, ay2, rdx, rdy, rdw, rdh, wb,
                ox1, oy1, ox2, oy2, okeep, alive):
    NP = ax1.shape[-1]
    wv = wb[:, 0:1]

    a_x1 = ax1[0]
    a_y1 = ay1[0]
    a_x2 = ax2[0]
    a_y2 = ay2[0]
    d_x = rdx[0]
    d_y = rdy[0]
    d_w = jnp.clip(rdw[0], -_MAX_RATIO, _MAX_RATIO)
    d_h = jnp.clip(rdh[0], -_MAX_RATIO, _MAX_RATIO)

    px = (a_x1 + a_x2) * 0.5
    py = (a_y1 + a_y2) * 0.5
    pw = a_x2 - a_x1
    ph = a_y2 - a_y1
    gx = px + pw * d_x
    gy = py + ph * d_y
    gw = pw * jnp.exp(d_w)
    gh = ph * jnp.exp(d_h)
    x1 = jnp.minimum(jnp.maximum(gx - gw * 0.5, 0.0), wv)
    y1 = jnp.minimum(jnp.maximum(gy - gh * 0.5, 0.0), wv)
    x2 = jnp.minimum(jnp.maximum(gx + gw * 0.5, 0.0), wv)
    y2 = jnp.minimum(jnp.maximum(gy + gh * 0.5, 0.0), wv)

    ox1[0] = x1
    oy1[0] = y1
    ox2[0] = x2
    oy2[0] = y2

    ok = (x2 > x1) & (y2 > y1)
    alive[...] = ok.astype(jnp.float32)

    area_all = (x2 - x1) * (y2 - y1)
    colid = jax.lax.broadcasted_iota(jnp.int32, (1, NP), 1)
    rj = jax.lax.broadcasted_iota(jnp.int32, (_C, _C), 0)
    ci = jax.lax.broadcasted_iota(jnp.int32, (_C, _C), 1)
    tri = rj < ci

    def chunk_body(k, carry):
        c0 = k * _C
        rx1 = ox1[0, :, pl.ds(c0, _C)]
        ry1 = oy1[0, :, pl.ds(c0, _C)]
        rx2 = ox2[0, :, pl.ds(c0, _C)]
        ry2 = oy2[0, :, pl.ds(c0, _C)]
        cx1 = jnp.transpose(rx1)
        cy1 = jnp.transpose(ry1)
        cx2 = jnp.transpose(rx2)
        cy2 = jnp.transpose(ry2)
        arear = (rx2 - rx1) * (ry2 - ry1)
        areac = jnp.transpose(arear)
        al_k = alive[:, pl.ds(c0, _C)] > 0.0

        ix1 = jnp.maximum(cx1, rx1)
        iy1 = jnp.maximum(cy1, ry1)
        ix2 = jnp.minimum(cx2, rx2)
        iy2 = jnp.minimum(cy2, ry2)
        inter = jnp.maximum(ix2 - ix1, 0.0) * jnp.maximum(iy2 - iy1, 0.0)
        iou = inter / (((areac + arear) - inter) + 1e-9)
        M = (iou > _NMS_THR) & tri

        def wcond(st):
            return (st[2] > 0) & (st[0] < _C)

        def wbody(st):
            t, krf, _ = st
            kcf = jnp.transpose(krf)
            sup = jnp.any(M & (kcf > 0.0), axis=0, keepdims=True)
            nkf = (al_k & jnp.logical_not(sup)).astype(jnp.float32)
            ch = jnp.any(nkf != krf)
            return (t + 1, nkf, jnp.where(ch, jnp.int32(1), jnp.int32(0)))

        _, keep_f, _ = jax.lax.while_loop(
            wcond, wbody, (jnp.int32(0), al_k.astype(jnp.float32), jnp.int32(1)))
        keep_row = keep_f > 0.0

        okeep[0, :, pl.ds(c0, _C)] = keep_f

        jx1 = jnp.maximum(cx1, x1)
        jy1 = jnp.maximum(cy1, y1)
        jx2 = jnp.minimum(cx2, x2)
        jy2 = jnp.minimum(cy2, y2)
        jinter = jnp.maximum(jx2 - jx1, 0.0) * jnp.maximum(jy2 - jy1, 0.0)
        jiou = jinter / (((areac + area_all) - jinter) + 1e-9)
        kc = jnp.transpose(keep_row)
        supa = jnp.any((jiou > _NMS_THR) & kc, axis=0, keepdims=True)
        supa = supa & (colid >= c0 + _C)
        alive[...] = jnp.where(supa, 0.0, alive[...])
        return carry

    jax.lax.fori_loop(0, NP // _C, chunk_body, 0)


def _flatten_level(cls_o, reg_o):
    B = cls_o.shape[0]
    logits = jnp.transpose(cls_o, (0, 2, 3, 1)).reshape(B, -1)
    deltas = jnp.transpose(reg_o, (0, 2, 3, 1)).reshape(B, -1, 4)
    return logits, deltas


def kernel(cls_out_0, cls_out_1, cls_out_2, reg_out_0, reg_out_1, reg_out_2, image_size):
    cls_outs = [cls_out_0, cls_out_1, cls_out_2]
    reg_outs = [reg_out_0, reg_out_1, reg_out_2]
    B = cls_out_0.shape[0]

    logit_sel, delta_sel, anchor_sel = [], [], []
    for c, r, s in zip(cls_outs, reg_outs, _STRIDES):
        anchors = jnp.asarray(_grid_anchors_np(c.shape[2], c.shape[3], s))
        logits, deltas = _flatten_level(c, r)
        k = min(_NUM_PRE_NMS, logits.shape[1])
        top_l, top_i = jax.lax.top_k(logits, k)
        logit_sel.append(top_l)
        delta_sel.append(jnp.take_along_axis(deltas, top_i[..., None], axis=1))
        anchor_sel.append(anchors[top_i])
    logits_c = jnp.concatenate(logit_sel, axis=1)
    deltas_c = jnp.concatenate(delta_sel, axis=1)
    anchors_c = jnp.concatenate(anchor_sel, axis=1)
    N = logits_c.shape[1]
    scores = jax.nn.sigmoid(logits_c)

    order = jnp.argsort(-scores, axis=-1)
    scores_s = jnp.take_along_axis(scores, order, axis=1)
    deltas_s = jnp.take_along_axis(deltas_c, order[..., None], axis=1)
    anchors_s = jnp.take_along_axis(anchors_c, order[..., None], axis=1)

    NP = ((N + _C - 1) // _C) * _C
    pad = NP - N
    deltas_s = jnp.pad(deltas_s, ((0, 0), (0, pad), (0, 0)))
    anchors_s = jnp.pad(anchors_s, ((0, 0), (0, pad), (0, 0)))

    def col(a, j):
        return a[:, :, j].reshape(B, 1, NP)

    w_f = jnp.full((1, 128), jnp.asarray(image_size, jnp.float32))

    in_specs = [pl.BlockSpec((1, 1, NP), lambda b: (b, 0, 0)) for _ in range(8)]
    in_specs.append(pl.BlockSpec((1, 128), lambda b: (0, 0)))
    out_specs = [pl.BlockSpec((1, 1, NP), lambda b: (b, 0, 0)) for _ in range(5)]
    outs = pl.pallas_call(
        _nms_kernel,
        grid=(B,),
        in_specs=in_specs,
        out_specs=out_specs,
        out_shape=[jax.ShapeDtypeStruct((B, 1, NP), jnp.float32)] * 5,
        scratch_shapes=[pltpu.VMEM((1, NP), jnp.float32)],
        interpret=_INTERPRET,
    )(col(anchors_s, 0), col(anchors_s, 1), col(anchors_s, 2), col(anchors_s, 3),
      col(deltas_s, 0), col(deltas_s, 1), col(deltas_s, 2), col(deltas_s, 3),
      w_f)

    x1o, y1o, x2o, y2o, keepf = outs
    boxes_s = jnp.stack([x1o[:, 0, :N], y1o[:, 0, :N], x2o[:, 0, :N], y2o[:, 0, :N]],
                        axis=-1)
    keep = keepf[:, 0, :N] > 0.5

    rank = jnp.cumsum(keep.astype(jnp.int32), axis=1) - 1
    valid = keep & (rank < _MAX_PER_IMG)
    pos = jnp.where(valid, rank, _MAX_PER_IMG)
    bi = jnp.arange(B)[:, None]
    out_b = jnp.zeros((B, _MAX_PER_IMG, 4), jnp.float32).at[bi, pos].set(
        boxes_s, mode='drop')
    out_s = jnp.zeros((B, _MAX_PER_IMG), jnp.float32).at[bi, pos].set(
        scores_s, mode='drop')
    out_l = jnp.zeros((B, _MAX_PER_IMG), jnp.int32)
    return out_b, out_s, out_l


# speedup vs baseline: 55.5485x; 1.4453x over previous
import numpy as np
import jax
import jax.numpy as jnp
from jax.experimental import pallas as pl
from jax.experimental.pallas import tpu as pltpu

_STRIDES = (8, 16, 32)
_RATIOS = np.array([0.5, 1.0, 2.0], dtype=np.float32)
_SCALE = 8.0
_NUM_PRE_NMS = 2000
_MAX_PER_IMG = 1000
_NMS_THR = 0.7
_MAX_RATIO = float(np.log(1000.0 / 16.0))
_C = 128
_INTERPRET = False


def _grid_anchors_np(H, W, stride):
    hr = np.sqrt(_RATIOS)
    wr = 1.0 / hr
    ws = stride * _SCALE * wr
    hs = stride * _SCALE * hr
    base = np.stack([-ws / 2.0, -hs / 2.0, ws / 2.0, hs / 2.0], axis=1).astype(np.float32)
    sx = np.arange(W, dtype=np.float32) * stride
    sy = np.arange(H, dtype=np.float32) * stride
    yy, xx = np.meshgrid(sy, sx, indexing='ij')
    shifts = np.stack([xx.ravel(), yy.ravel(), xx.ravel(), yy.ravel()], axis=1)
    return (shifts[:, None, :] + base[None, :, :]).reshape(-1, 4).astype(np.float32)


def _nms_kernel(ax1, ay1, ax2, ay2, rdx, rdy, rdw, rdh, wb,
                ox1, oy1, ox2, oy2, okeep, alive, cnt):
    NP = ax1.shape[-1]
    wv = wb[:, 0:1]

    a_x1 = ax1[0]
    a_y1 = ay1[0]
    a_x2 = ax2[0]
    a_y2 = ay2[0]
    d_x = rdx[0]
    d_y = rdy[0]
    d_w = jnp.clip(rdw[0], -_MAX_RATIO, _MAX_RATIO)
    d_h = jnp.clip(rdh[0], -_MAX_RATIO, _MAX_RATIO)

    px = (a_x1 + a_x2) * 0.5
    py = (a_y1 + a_y2) * 0.5
    pw = a_x2 - a_x1
    ph = a_y2 - a_y1
    gx = px + pw * d_x
    gy = py + ph * d_y
    gw = pw * jnp.exp(d_w)
    gh = ph * jnp.exp(d_h)
    x1 = jnp.minimum(jnp.maximum(gx - gw * 0.5, 0.0), wv)
    y1 = jnp.minimum(jnp.maximum(gy - gh * 0.5, 0.0), wv)
    x2 = jnp.minimum(jnp.maximum(gx + gw * 0.5, 0.0), wv)
    y2 = jnp.minimum(jnp.maximum(gy + gh * 0.5, 0.0), wv)

    ox1[0] = x1
    oy1[0] = y1
    ox2[0] = x2
    oy2[0] = y2

    ok = (x2 > x1) & (y2 > y1)
    alive[...] = ok.astype(jnp.float32)
    okeep[0] = jnp.zeros((1, NP), jnp.float32)
    cnt[0] = jnp.int32(0)

    rj = jax.lax.broadcasted_iota(jnp.int32, (_C, _C), 0)
    ci = jax.lax.broadcasted_iota(jnp.int32, (_C, _C), 1)
    tri = rj < ci

    def chunk_body(k, carry):
        @pl.when(cnt[0] < _MAX_PER_IMG)
        def _process():
            _do_chunk(k)
        return carry

    def _do_chunk(k):
        c0 = k * _C
        rx1 = ox1[0, :, pl.ds(c0, _C)]
        ry1 = oy1[0, :, pl.ds(c0, _C)]
        rx2 = ox2[0, :, pl.ds(c0, _C)]
        ry2 = oy2[0, :, pl.ds(c0, _C)]
        cx1 = jnp.transpose(rx1)
        cy1 = jnp.transpose(ry1)
        cx2 = jnp.transpose(rx2)
        cy2 = jnp.transpose(ry2)
        arear = (rx2 - rx1) * (ry2 - ry1)
        areac = jnp.transpose(arear)
        al_k = alive[:, pl.ds(c0, _C)] > 0.0

        ix1 = jnp.maximum(cx1, rx1)
        iy1 = jnp.maximum(cy1, ry1)
        ix2 = jnp.minimum(cx2, rx2)
        iy2 = jnp.minimum(cy2, ry2)
        inter = jnp.maximum(ix2 - ix1, 0.0) * jnp.maximum(iy2 - iy1, 0.0)
        iou = inter / (((areac + arear) - inter) + 1e-9)
        M = (iou > _NMS_THR) & tri

        def wcond(st):
            return (st[2] > 0) & (st[0] < _C)

        def wbody(st):
            t, krf, _ = st
            kcf = jnp.transpose(krf)
            sup = jnp.any(M & (kcf > 0.0), axis=0, keepdims=True)
            nkf = (al_k & jnp.logical_not(sup)).astype(jnp.float32)
            ch = jnp.any(nkf != krf)
            return (t + 1, nkf, jnp.where(ch, jnp.int32(1), jnp.int32(0)))

        _, keep_f, _ = jax.lax.while_loop(
            wcond, wbody, (jnp.int32(0), al_k.astype(jnp.float32), jnp.int32(1)))
        keep_row = keep_f > 0.0

        okeep[0, :, pl.ds(c0, _C)] = keep_f
        cnt[0] = cnt[0] + jnp.sum(keep_f).astype(jnp.int32)

        kc = jnp.transpose(keep_row)

        def cross_body(k2, carry2):
            c2 = k2 * _C
            tx1 = ox1[0, :, pl.ds(c2, _C)]
            ty1 = oy1[0, :, pl.ds(c2, _C)]
            tx2 = ox2[0, :, pl.ds(c2, _C)]
            ty2 = oy2[0, :, pl.ds(c2, _C)]
            tarea = (tx2 - tx1) * (ty2 - ty1)
            jx1 = jnp.maximum(cx1, tx1)
            jy1 = jnp.maximum(cy1, ty1)
            jx2 = jnp.minimum(cx2, tx2)
            jy2 = jnp.minimum(cy2, ty2)
            jinter = jnp.maximum(jx2 - jx1, 0.0) * jnp.maximum(jy2 - jy1, 0.0)
            jiou = jinter / (((areac + tarea) - jinter) + 1e-9)
            supa = jnp.any((jiou > _NMS_THR) & kc, axis=0, keepdims=True)
            alv = alive[:, pl.ds(c2, _C)]
            alive[:, pl.ds(c2, _C)] = jnp.where(supa, 0.0, alv)
            return carry2

        jax.lax.fori_loop(k + 1, NP // _C, cross_body, 0)

    jax.lax.fori_loop(0, NP // _C, chunk_body, 0)


def _flatten_level(cls_o, reg_o):
    B = cls_o.shape[0]
    logits = jnp.transpose(cls_o, (0, 2, 3, 1)).reshape(B, -1)
    deltas = jnp.transpose(reg_o, (0, 2, 3, 1)).reshape(B, -1, 4)
    return logits, deltas


def kernel(cls_out_0, cls_out_1, cls_out_2, reg_out_0, reg_out_1, reg_out_2, image_size):
    cls_outs = [cls_out_0, cls_out_1, cls_out_2]
    reg_outs = [reg_out_0, reg_out_1, reg_out_2]
    B = cls_out_0.shape[0]

    logit_sel, da_sel = [], []
    for c, r, s in zip(cls_outs, reg_outs, _STRIDES):
        anchors = jnp.asarray(_grid_anchors_np(c.shape[2], c.shape[3], s))
        logits, deltas = _flatten_level(c, r)
        da = jnp.concatenate(
            [deltas, jnp.broadcast_to(anchors[None], (B,) + anchors.shape)], axis=-1)
        k = min(_NUM_PRE_NMS, logits.shape[1])
        top_l, top_i = jax.lax.top_k(logits, k)
        logit_sel.append(top_l)
        da_sel.append(jnp.take_along_axis(da, top_i[..., None], axis=1))
    logits_c = jnp.concatenate(logit_sel, axis=1)
    da_c = jnp.concatenate(da_sel, axis=1)
    N = logits_c.shape[1]
    scores = jax.nn.sigmoid(logits_c)

    order = jnp.argsort(-scores, axis=-1)
    das_c = jnp.concatenate([da_c, scores[..., None]], axis=-1)
    das_s = jnp.take_along_axis(das_c, order[..., None], axis=1)
    deltas_s = das_s[..., 0:4]
    anchors_s = das_s[..., 4:8]
    scores_s = das_s[..., 8]

    NP = ((N + _C - 1) // _C) * _C
    pad = NP - N
    deltas_s = jnp.pad(deltas_s, ((0, 0), (0, pad), (0, 0)))
    anchors_s = jnp.pad(anchors_s, ((0, 0), (0, pad), (0, 0)))

    def col(a, j):
        return a[:, :, j].reshape(B, 1, NP)

    w_f = jnp.full((1, 128), jnp.asarray(image_size, jnp.float32))

    in_specs = [pl.BlockSpec((1, 1, NP), lambda b: (b, 0, 0)) for _ in range(8)]
    in_specs.append(pl.BlockSpec((1, 128), lambda b: (0, 0)))
    out_specs = [pl.BlockSpec((1, 1, NP), lambda b: (b, 0, 0)) for _ in range(5)]
    outs = pl.pallas_call(
        _nms_kernel,
        grid=(B,),
        in_specs=in_specs,
        out_specs=out_specs,
        out_shape=[jax.ShapeDtypeStruct((B, 1, NP), jnp.float32)] * 5,
        scratch_shapes=[pltpu.VMEM((1, NP), jnp.float32),
                        pltpu.SMEM((1,), jnp.int32)],
        interpret=_INTERPRET,
    )(col(anchors_s, 0), col(anchors_s, 1), col(anchors_s, 2), col(anchors_s, 3),
      col(deltas_s, 0), col(deltas_s, 1), col(deltas_s, 2), col(deltas_s, 3),
      w_f)

    x1o, y1o, x2o, y2o, keepf = outs
    boxes_s = jnp.stack([x1o[:, 0, :N], y1o[:, 0, :N], x2o[:, 0, :N], y2o[:, 0, :N]],
                        axis=-1)
    keep = keepf[:, 0, :N] > 0.5

    rank = jnp.cumsum(keep.astype(jnp.int32), axis=1) - 1
    valid = keep & (rank < _MAX_PER_IMG)
    pos = jnp.where(valid, rank, _MAX_PER_IMG)
    bi = jnp.arange(B)[:, None]
    out_b = jnp.zeros((B, _MAX_PER_IMG, 4), jnp.float32).at[bi, pos].set(
        boxes_s, mode='drop')
    out_s = jnp.zeros((B, _MAX_PER_IMG), jnp.float32).at[bi, pos].set(
        scores_s, mode='drop')
    out_l = jnp.zeros((B, _MAX_PER_IMG), jnp.int32)
    return out_b, out_s, out_l


# speedup vs baseline: 56.4242x; 1.0158x over previous
import numpy as np
import jax
import jax.numpy as jnp
from jax.experimental import pallas as pl
from jax.experimental.pallas import tpu as pltpu

_STRIDES = (8, 16, 32)
_RATIOS = np.array([0.5, 1.0, 2.0], dtype=np.float32)
_SCALE = 8.0
_NUM_PRE_NMS = 2000
_MAX_PER_IMG = 1000
_NMS_THR = 0.7
_MAX_RATIO = float(np.log(1000.0 / 16.0))
_C = 128
_INTERPRET = False


def _grid_anchors_np(H, W, stride):
    hr = np.sqrt(_RATIOS)
    wr = 1.0 / hr
    ws = stride * _SCALE * wr
    hs = stride * _SCALE * hr
    base = np.stack([-ws / 2.0, -hs / 2.0, ws / 2.0, hs / 2.0], axis=1).astype(np.float32)
    sx = np.arange(W, dtype=np.float32) * stride
    sy = np.arange(H, dtype=np.float32) * stride
    yy, xx = np.meshgrid(sy, sx, indexing='ij')
    shifts = np.stack([xx.ravel(), yy.ravel(), xx.ravel(), yy.ravel()], axis=1)
    return (shifts[:, None, :] + base[None, :, :]).reshape(-1, 4).astype(np.float32)


def _nms_kernel(ax1, ay1, ax2, ay2, rdx, rdy, rdw, rdh, wb,
                ox1, oy1, ox2, oy2, okeep, alive, cnt):
    NP = ax1.shape[-1]
    wv = wb[:, 0:1]

    a_x1 = ax1[0]
    a_y1 = ay1[0]
    a_x2 = ax2[0]
    a_y2 = ay2[0]
    d_x = rdx[0]
    d_y = rdy[0]
    d_w = jnp.clip(rdw[0], -_MAX_RATIO, _MAX_RATIO)
    d_h = jnp.clip(rdh[0], -_MAX_RATIO, _MAX_RATIO)

    px = (a_x1 + a_x2) * 0.5
    py = (a_y1 + a_y2) * 0.5
    pw = a_x2 - a_x1
    ph = a_y2 - a_y1
    gx = px + pw * d_x
    gy = py + ph * d_y
    gw = pw * jnp.exp(d_w)
    gh = ph * jnp.exp(d_h)
    x1 = jnp.minimum(jnp.maximum(gx - gw * 0.5, 0.0), wv)
    y1 = jnp.minimum(jnp.maximum(gy - gh * 0.5, 0.0), wv)
    x2 = jnp.minimum(jnp.maximum(gx + gw * 0.5, 0.0), wv)
    y2 = jnp.minimum(jnp.maximum(gy + gh * 0.5, 0.0), wv)

    ox1[0] = x1
    oy1[0] = y1
    ox2[0] = x2
    oy2[0] = y2

    ok = (x2 > x1) & (y2 > y1)
    alive[...] = ok.astype(jnp.float32)
    okeep[0] = jnp.zeros((1, NP), jnp.float32)
    cnt[0] = jnp.int32(0)

    rj = jax.lax.broadcasted_iota(jnp.int32, (_C, _C), 0)
    ci = jax.lax.broadcasted_iota(jnp.int32, (_C, _C), 1)
    tri = rj < ci

    def chunk_body(k, carry):
        @pl.when(cnt[0] < _MAX_PER_IMG)
        def _process():
            _do_chunk(k)
        return carry

    def _do_chunk(k):
        c0 = k * _C
        rx1 = ox1[0, :, pl.ds(c0, _C)]
        ry1 = oy1[0, :, pl.ds(c0, _C)]
        rx2 = ox2[0, :, pl.ds(c0, _C)]
        ry2 = oy2[0, :, pl.ds(c0, _C)]
        cx1 = jnp.transpose(rx1)
        cy1 = jnp.transpose(ry1)
        cx2 = jnp.transpose(rx2)
        cy2 = jnp.transpose(ry2)
        arear = (rx2 - rx1) * (ry2 - ry1)
        areac = jnp.transpose(arear)
        al_k = alive[:, pl.ds(c0, _C)] > 0.0

        ix1 = jnp.maximum(cx1, rx1)
        iy1 = jnp.maximum(cy1, ry1)
        ix2 = jnp.minimum(cx2, rx2)
        iy2 = jnp.minimum(cy2, ry2)
        inter = jnp.maximum(ix2 - ix1, 0.0) * jnp.maximum(iy2 - iy1, 0.0)
        iou = inter / (((areac + arear) - inter) + 1e-9)
        M = (iou > _NMS_THR) & tri

        def wcond(st):
            return (st[2] > 0) & (st[0] < _C)

        def wbody(st):
            t, krf, _ = st
            kcf = jnp.transpose(krf)
            sup = jnp.any(M & (kcf > 0.0), axis=0, keepdims=True)
            nkf = (al_k & jnp.logical_not(sup)).astype(jnp.float32)
            ch = jnp.any(nkf != krf)
            return (t + 1, nkf, jnp.where(ch, jnp.int32(1), jnp.int32(0)))

        _, keep_f, _ = jax.lax.while_loop(
            wcond, wbody, (jnp.int32(0), al_k.astype(jnp.float32), jnp.int32(1)))
        keep_row = keep_f > 0.0

        okeep[0, :, pl.ds(c0, _C)] = keep_f
        cnt[0] = cnt[0] + jnp.sum(keep_f).astype(jnp.int32)

        kc = jnp.transpose(keep_row)

        def cross_body(k2, carry2):
            c2 = k2 * _C
            tx1 = ox1[0, :, pl.ds(c2, _C)]
            ty1 = oy1[0, :, pl.ds(c2, _C)]
            tx2 = ox2[0, :, pl.ds(c2, _C)]
            ty2 = oy2[0, :, pl.ds(c2, _C)]
            tarea = (tx2 - tx1) * (ty2 - ty1)
            jx1 = jnp.maximum(cx1, tx1)
            jy1 = jnp.maximum(cy1, ty1)
            jx2 = jnp.minimum(cx2, tx2)
            jy2 = jnp.minimum(cy2, ty2)
            jinter = jnp.maximum(jx2 - jx1, 0.0) * jnp.maximum(jy2 - jy1, 0.0)
            jiou = jinter / (((areac + tarea) - jinter) + 1e-9)
            supa = jnp.any((jiou > _NMS_THR) & kc, axis=0, keepdims=True)
            alv = alive[:, pl.ds(c2, _C)]
            alive[:, pl.ds(c2, _C)] = jnp.where(supa, 0.0, alv)
            return carry2

        jax.lax.fori_loop(k + 1, NP // _C, cross_body, 0)

    jax.lax.fori_loop(0, NP // _C, chunk_body, 0)


def _flatten_level(cls_o, reg_o):
    B = cls_o.shape[0]
    logits = jnp.transpose(cls_o, (0, 2, 3, 1)).reshape(B, -1)
    deltas = jnp.transpose(reg_o, (0, 2, 3, 1)).reshape(B, -1, 4)
    return logits, deltas


def kernel(cls_out_0, cls_out_1, cls_out_2, reg_out_0, reg_out_1, reg_out_2, image_size):
    cls_outs = [cls_out_0, cls_out_1, cls_out_2]
    reg_outs = [reg_out_0, reg_out_1, reg_out_2]
    B = cls_out_0.shape[0]

    logit_sel, da_sel = [], []
    for c, r, s in zip(cls_outs, reg_outs, _STRIDES):
        anchors = jnp.asarray(_grid_anchors_np(c.shape[2], c.shape[3], s))
        logits, deltas = _flatten_level(c, r)
        da = jnp.concatenate(
            [deltas, jnp.broadcast_to(anchors[None], (B,) + anchors.shape)], axis=-1)
        k = min(_NUM_PRE_NMS, logits.shape[1])
        top_l, top_i = jax.lax.top_k(logits, k)
        logit_sel.append(top_l)
        da_sel.append(jnp.take_along_axis(da, top_i[..., None], axis=1))
    logits_c = jnp.concatenate(logit_sel, axis=1)
    da_c = jnp.concatenate(da_sel, axis=1)
    N = logits_c.shape[1]
    scores = jax.nn.sigmoid(logits_c)

    scores_s, order = jax.lax.top_k(scores, N)
    das_s = jnp.take_along_axis(da_c, order[..., None], axis=1)
    deltas_s = das_s[..., 0:4]
    anchors_s = das_s[..., 4:8]

    NP = ((N + _C - 1) // _C) * _C
    pad = NP - N
    deltas_s = jnp.pad(deltas_s, ((0, 0), (0, pad), (0, 0)))
    anchors_s = jnp.pad(anchors_s, ((0, 0), (0, pad), (0, 0)))

    def col(a, j):
        return a[:, :, j].reshape(B, 1, NP)

    w_f = jnp.full((1, 128), jnp.asarray(image_size, jnp.float32))

    in_specs = [pl.BlockSpec((1, 1, NP), lambda b: (b, 0, 0)) for _ in range(8)]
    in_specs.append(pl.BlockSpec((1, 128), lambda b: (0, 0)))
    out_specs = [pl.BlockSpec((1, 1, NP), lambda b: (b, 0, 0)) for _ in range(5)]
    outs = pl.pallas_call(
        _nms_kernel,
        grid=(B,),
        in_specs=in_specs,
        out_specs=out_specs,
        out_shape=[jax.ShapeDtypeStruct((B, 1, NP), jnp.float32)] * 5,
        scratch_shapes=[pltpu.VMEM((1, NP), jnp.float32),
                        pltpu.SMEM((1,), jnp.int32)],
        interpret=_INTERPRET,
    )(col(anchors_s, 0), col(anchors_s, 1), col(anchors_s, 2), col(anchors_s, 3),
      col(deltas_s, 0), col(deltas_s, 1), col(deltas_s, 2), col(deltas_s, 3),
      w_f)

    x1o, y1o, x2o, y2o, keepf = outs
    boxes_s = jnp.stack([x1o[:, 0, :N], y1o[:, 0, :N], x2o[:, 0, :N], y2o[:, 0, :N]],
                        axis=-1)
    keep = keepf[:, 0, :N] > 0.5

    rank = jnp.cumsum(keep.astype(jnp.int32), axis=1) - 1
    valid = keep & (rank < _MAX_PER_IMG)
    pos = jnp.where(valid, rank, _MAX_PER_IMG)
    bi = jnp.arange(B)[:, None]
    out_b = jnp.zeros((B, _MAX_PER_IMG, 4), jnp.float32).at[bi, pos].set(
        boxes_s, mode='drop')
    out_s = jnp.zeros((B, _MAX_PER_IMG), jnp.float32).at[bi, pos].set(
        scores_s, mode='drop')
    out_l = jnp.zeros((B, _MAX_PER_IMG), jnp.int32)
    return out_b, out_s, out_l
